# baseline (speedup 1.0000x reference)
import functools
import math

import jax
import jax.numpy as jnp
from jax.experimental import pallas as pl

H = 16


def _matmul_kernel(x_ref, w_ref, b_ref, o_ref):
    o_ref[...] = (
        jnp.dot(x_ref[...], w_ref[...], preferred_element_type=jnp.float32)
        + b_ref[...]
    )


def _matmul(x, W, b, bm=256, bn=512):
    M, K = x.shape
    N = W.shape[1]
    return pl.pallas_call(
        _matmul_kernel,
        grid=(M // bm, N // bn),
        in_specs=[
            pl.BlockSpec((bm, K), lambda i, j: (i, 0)),
            pl.BlockSpec((K, bn), lambda i, j: (0, j)),
            pl.BlockSpec((1, bn), lambda i, j: (0, j)),
        ],
        out_specs=pl.BlockSpec((bm, bn), lambda i, j: (i, j)),
        out_shape=jax.ShapeDtypeStruct((M, N), jnp.float32),
    )(x, W, b.reshape(1, N))


def _attn_kernel(q_ref, k_ref, v_ref, o_ref, *, scale):
    q = q_ref[0]
    k = k_ref[0]
    v = v_ref[0]
    s = jax.lax.dot_general(
        q, k, (((1,), (1,)), ((), ())), preferred_element_type=jnp.float32
    ) * scale
    m = jnp.max(s, axis=-1, keepdims=True)
    p = jnp.exp(s - m)
    p = p / jnp.sum(p, axis=-1, keepdims=True)
    o_ref[0] = jnp.dot(p, v, preferred_element_type=jnp.float32)


def _attention(qp, kp, vp, bq=256):
    B, S, D = qp.shape
    DK = D // H
    scale = 1.0 / math.sqrt(DK)
    return pl.pallas_call(
        functools.partial(_attn_kernel, scale=scale),
        grid=(B, H, S // bq),
        in_specs=[
            pl.BlockSpec((1, bq, DK), lambda b, h, i: (b, i, h)),
            pl.BlockSpec((1, S, DK), lambda b, h, i: (b, 0, h)),
            pl.BlockSpec((1, S, DK), lambda b, h, i: (b, 0, h)),
        ],
        out_specs=pl.BlockSpec((1, bq, DK), lambda b, h, i: (b, i, h)),
        out_shape=jax.ShapeDtypeStruct((B, S, D), jnp.float32),
    )(qp, kp, vp)


@jax.jit
def kernel(query, key, value, Wq, bq, Wk, bk, Wv, bv, Wo, bo):
    B, S, D = query.shape
    q2 = query.reshape(B * S, D)
    k2 = key.reshape(B * S, D)
    v2 = value.reshape(B * S, D)

    qp = _matmul(q2, Wq, bq).reshape(B, S, D)
    kp = _matmul(k2, Wk, bk).reshape(B, S, D)
    vp = _matmul(v2, Wv, bv).reshape(B, S, D)

    ctx = _attention(qp, kp, vp)

    out = _matmul(ctx.reshape(B * S, D), Wo, bo)
    return out.reshape(B, S, D)


# speedup vs baseline: 1.1959x; 1.1959x over previous
import functools
import math

import jax
import jax.numpy as jnp
from jax.experimental import pallas as pl

H = 16


def _matmul_kernel(x_ref, w_ref, b_ref, o_ref):
    acc = jnp.dot(x_ref[...], w_ref[...], preferred_element_type=jnp.float32)
    o_ref[...] = (acc + b_ref[...]).astype(o_ref.dtype)


def _matmul(x, W, b, out_dtype, bm=256, bn=512):
    M, K = x.shape
    N = W.shape[1]
    return pl.pallas_call(
        _matmul_kernel,
        grid=(M // bm, N // bn),
        in_specs=[
            pl.BlockSpec((bm, K), lambda i, j: (i, 0)),
            pl.BlockSpec((K, bn), lambda i, j: (0, j)),
            pl.BlockSpec((1, bn), lambda i, j: (0, j)),
        ],
        out_specs=pl.BlockSpec((bm, bn), lambda i, j: (i, j)),
        out_shape=jax.ShapeDtypeStruct((M, N), out_dtype),
    )(x, W, b.reshape(1, N))


def _attn_kernel(q_ref, k_ref, v_ref, o_ref, *, scale):
    q = q_ref[0]
    k = k_ref[0]
    v = v_ref[0]
    s = jax.lax.dot_general(
        q, k, (((1,), (1,)), ((), ())), preferred_element_type=jnp.float32
    ) * scale
    m = jnp.max(s, axis=-1, keepdims=True)
    p = jnp.exp(s - m)
    l = jnp.sum(p, axis=-1, keepdims=True)
    ctx = jnp.dot(p.astype(jnp.bfloat16), v, preferred_element_type=jnp.float32)
    o_ref[0] = (ctx / l).astype(o_ref.dtype)


def _attention(qp, kp, vp, bq=256):
    B, S, D = qp.shape
    DK = D // H
    scale = 1.0 / math.sqrt(DK)
    return pl.pallas_call(
        functools.partial(_attn_kernel, scale=scale),
        grid=(B, H, S // bq),
        in_specs=[
            pl.BlockSpec((1, bq, DK), lambda b, h, i: (b, i, h)),
            pl.BlockSpec((1, S, DK), lambda b, h, i: (b, 0, h)),
            pl.BlockSpec((1, S, DK), lambda b, h, i: (b, 0, h)),
        ],
        out_specs=pl.BlockSpec((1, bq, DK), lambda b, h, i: (b, i, h)),
        out_shape=jax.ShapeDtypeStruct((B, S, D), jnp.bfloat16),
    )(qp, kp, vp)


@jax.jit
def kernel(query, key, value, Wq, bq, Wk, bk, Wv, bv, Wo, bo):
    B, S, D = query.shape
    bf = jnp.bfloat16
    q2 = query.reshape(B * S, D).astype(bf)
    k2 = key.reshape(B * S, D).astype(bf)
    v2 = value.reshape(B * S, D).astype(bf)

    qp = _matmul(q2, Wq.astype(bf), bq, bf).reshape(B, S, D)
    kp = _matmul(k2, Wk.astype(bf), bk, bf).reshape(B, S, D)
    vp = _matmul(v2, Wv.astype(bf), bv, bf).reshape(B, S, D)

    ctx = _attention(qp, kp, vp)

    out = _matmul(ctx.reshape(B * S, D), Wo.astype(bf), bo, jnp.float32)
    return out.reshape(B, S, D)


# speedup vs baseline: 1.4027x; 1.1729x over previous
import functools
import math

import jax
import jax.numpy as jnp
from jax.experimental import pallas as pl

H = 16


def _matmul_kernel(x_ref, w_ref, b_ref, o_ref):
    acc = jnp.dot(x_ref[...], w_ref[...], preferred_element_type=jnp.float32)
    o_ref[...] = (acc + b_ref[...]).astype(o_ref.dtype)


def _matmul(x, W, b, out_dtype, bm=256):
    M, K = x.shape
    N = W.shape[1]
    return pl.pallas_call(
        _matmul_kernel,
        grid=(M // bm,),
        in_specs=[
            pl.BlockSpec((bm, K), lambda i: (i, 0)),
            pl.BlockSpec((K, N), lambda i: (0, 0)),
            pl.BlockSpec((1, N), lambda i: (0, 0)),
        ],
        out_specs=pl.BlockSpec((bm, N), lambda i: (i, 0)),
        out_shape=jax.ShapeDtypeStruct((M, N), out_dtype),
    )(x, W, b.reshape(1, N))


def _attn_kernel(q_ref, k_ref, v_ref, o_ref, *, scale):
    q = q_ref[0]
    k = k_ref[0]
    v = v_ref[0]
    s = jax.lax.dot_general(
        q, k, (((1,), (1,)), ((), ())), preferred_element_type=jnp.float32
    ) * scale
    m = jnp.max(s, axis=-1, keepdims=True)
    p = jnp.exp(s - m)
    l = jnp.sum(p, axis=-1, keepdims=True)
    ctx = jnp.dot(p.astype(jnp.bfloat16), v, preferred_element_type=jnp.float32)
    o_ref[0] = (ctx / l).astype(o_ref.dtype)


def _attention(qp, kp, vp, bq=512):
    B, S, D = qp.shape
    DK = D // H
    scale = 1.0 / math.sqrt(DK)
    return pl.pallas_call(
        functools.partial(_attn_kernel, scale=scale),
        grid=(B, H, S // bq),
        in_specs=[
            pl.BlockSpec((1, bq, DK), lambda b, h, i: (b, i, h)),
            pl.BlockSpec((1, S, DK), lambda b, h, i: (b, 0, h)),
            pl.BlockSpec((1, S, DK), lambda b, h, i: (b, 0, h)),
        ],
        out_specs=pl.BlockSpec((1, bq, DK), lambda b, h, i: (b, i, h)),
        out_shape=jax.ShapeDtypeStruct((B, S, D), jnp.bfloat16),
    )(qp, kp, vp)


@jax.jit
def kernel(query, key, value, Wq, bq, Wk, bk, Wv, bv, Wo, bo):
    B, S, D = query.shape
    bf = jnp.bfloat16
    q2 = query.reshape(B * S, D).astype(bf)
    k2 = key.reshape(B * S, D).astype(bf)
    v2 = value.reshape(B * S, D).astype(bf)

    qp = _matmul(q2, Wq.astype(bf), bq, bf).reshape(B, S, D)
    kp = _matmul(k2, Wk.astype(bf), bk, bf).reshape(B, S, D)
    vp = _matmul(v2, Wv.astype(bf), bv, bf).reshape(B, S, D)

    ctx = _attention(qp, kp, vp)

    out = _matmul(ctx.reshape(B * S, D), Wo.astype(bf), bo, jnp.float32)
    return out.reshape(B, S, D)


# speedup vs baseline: 1.5160x; 1.0808x over previous
import functools
import math

import jax
import jax.numpy as jnp
from jax.experimental import pallas as pl

H = 16


def _matmul_kernel(x_ref, w_ref, b_ref, o_ref):
    acc = jnp.dot(x_ref[...], w_ref[...], preferred_element_type=jnp.float32)
    o_ref[...] = (acc + b_ref[...]).astype(o_ref.dtype)


def _matmul(x, W, b, out_dtype, bm=256):
    M, K = x.shape
    N = W.shape[1]
    return pl.pallas_call(
        _matmul_kernel,
        grid=(M // bm,),
        in_specs=[
            pl.BlockSpec((bm, K), lambda i: (i, 0)),
            pl.BlockSpec((K, N), lambda i: (0, 0)),
            pl.BlockSpec((1, N), lambda i: (0, 0)),
        ],
        out_specs=pl.BlockSpec((bm, N), lambda i: (i, 0)),
        out_shape=jax.ShapeDtypeStruct((M, N), out_dtype),
    )(x, W, b.reshape(1, N))


def _attn_kernel(q_ref, k_ref, v_ref, o_ref, *, nc, C):
    q = q_ref[0]
    ss = []
    for j in range(nc):
        kj = k_ref[0, j * C:(j + 1) * C, :]
        ss.append(
            jax.lax.dot_general(
                q, kj, (((1,), (1,)), ((), ())),
                preferred_element_type=jnp.float32,
            )
        )
    m = functools.reduce(
        jnp.maximum, [jnp.max(s, axis=-1, keepdims=True) for s in ss]
    )
    acc = None
    l = None
    for j in range(nc):
        p = jnp.exp(ss[j] - m)
        lj = jnp.sum(p, axis=-1, keepdims=True)
        dj = jnp.dot(
            p.astype(jnp.bfloat16),
            v_ref[0, j * C:(j + 1) * C, :],
            preferred_element_type=jnp.float32,
        )
        acc = dj if acc is None else acc + dj
        l = lj if l is None else l + lj
    o_ref[0] = (acc / l).astype(o_ref.dtype)


def _attention(qp, kp, vp, bq=512, C=512):
    B, S, D = qp.shape
    DK = D // H
    return pl.pallas_call(
        functools.partial(_attn_kernel, nc=S // C, C=C),
        grid=(B, H, S // bq),
        in_specs=[
            pl.BlockSpec((1, bq, DK), lambda b, h, i: (b, i, h)),
            pl.BlockSpec((1, S, DK), lambda b, h, i: (b, 0, h)),
            pl.BlockSpec((1, S, DK), lambda b, h, i: (b, 0, h)),
        ],
        out_specs=pl.BlockSpec((1, bq, DK), lambda b, h, i: (b, i, h)),
        out_shape=jax.ShapeDtypeStruct((B, S, D), jnp.bfloat16),
    )(qp, kp, vp)


@jax.jit
def kernel(query, key, value, Wq, bq, Wk, bk, Wv, bv, Wo, bo):
    B, S, D = query.shape
    bf = jnp.bfloat16
    q2 = query.reshape(B * S, D).astype(bf)
    k2 = key.reshape(B * S, D).astype(bf)
    v2 = value.reshape(B * S, D).astype(bf)

    scale = 1.0 / math.sqrt(D // H)
    qp = _matmul(q2, (Wq * scale).astype(bf), bq * scale, bf).reshape(B, S, D)
    kp = _matmul(k2, Wk.astype(bf), bk, bf).reshape(B, S, D)
    vp = _matmul(v2, Wv.astype(bf), bv, bf).reshape(B, S, D)

    ctx = _attention(qp, kp, vp)

    out = _matmul(ctx.reshape(B * S, D), Wo.astype(bf), bo, jnp.float32)
    return out.reshape(B, S, D)


# speedup vs baseline: 2.0334x; 1.3412x over previous
import functools
import math

import jax
import jax.numpy as jnp
from jax.experimental import pallas as pl

H = 16


def _qkv_kernel(x1_ref, x2_ref, x3_ref, w1_ref, w2_ref, w3_ref,
                b1_ref, b2_ref, b3_ref, o1_ref, o2_ref, o3_ref):
    for x_ref, w_ref, b_ref, o_ref in (
        (x1_ref, w1_ref, b1_ref, o1_ref),
        (x2_ref, w2_ref, b2_ref, o2_ref),
        (x3_ref, w3_ref, b3_ref, o3_ref),
    ):
        xb = x_ref[...].astype(jnp.bfloat16)
        acc = jnp.dot(xb, w_ref[...], preferred_element_type=jnp.float32)
        o_ref[...] = (acc + b_ref[...]).astype(jnp.bfloat16)


def _qkv_proj(q2, k2, v2, Wqs, bqs, Wk, bk, Wv, bv, bm=256):
    M, K = q2.shape
    N = Wqs.shape[1]
    bf = jnp.bfloat16
    blk_x = pl.BlockSpec((bm, K), lambda i: (i, 0))
    blk_w = pl.BlockSpec((K, N), lambda i: (0, 0))
    blk_b = pl.BlockSpec((1, N), lambda i: (0, 0))
    blk_o = pl.BlockSpec((bm, N), lambda i: (i, 0))
    out = jax.ShapeDtypeStruct((M, N), bf)
    return pl.pallas_call(
        _qkv_kernel,
        grid=(M // bm,),
        in_specs=[blk_x, blk_x, blk_x, blk_w, blk_w, blk_w,
                  blk_b, blk_b, blk_b],
        out_specs=[blk_o, blk_o, blk_o],
        out_shape=[out, out, out],
    )(q2, k2, v2, Wqs, Wk, Wv,
      bqs.reshape(1, N), bk.reshape(1, N), bv.reshape(1, N))


def _attn_out_kernel(q_ref, k_ref, v_ref, wo_ref, bo_ref, o_ref, ctx_ref):
    D = q_ref.shape[1]
    DK = D // H
    tokens = []
    for h in range(H):
        lo, hi = h * DK, (h + 1) * DK
        qh = q_ref[:, lo:hi]
        if h >= 2:
            qh = (qh.astype(jnp.float32)
                  + jnp.minimum(tokens[h - 2], 0.0)).astype(jnp.bfloat16)
        s = jax.lax.dot_general(
            qh, k_ref[:, lo:hi], (((1,), (1,)), ((), ())),
            preferred_element_type=jnp.float32,
        )
        m = jnp.max(s, axis=-1, keepdims=True)
        p = jnp.exp2(s - m)
        l = jnp.sum(p, axis=-1, keepdims=True)
        d = jnp.dot(p.astype(jnp.bfloat16), v_ref[:, lo:hi],
                    preferred_element_type=jnp.float32)
        ctx_ref[:, lo:hi] = (d / l).astype(jnp.bfloat16)
        tokens.append(l)
    o_ref[...] = (
        jnp.dot(ctx_ref[...], wo_ref[...], preferred_element_type=jnp.float32)
        + bo_ref[...]
    )


def _attn_out(qp, kp, vp, Wob, bo, B=2, bq=256):
    from jax.experimental.pallas import tpu as pltpu

    BS, D = qp.shape
    S = BS // B
    nq = S // bq
    outs = []
    for b in range(B):
        outs.append(pl.pallas_call(
            _attn_out_kernel,
            grid=(nq,),
            in_specs=[
                pl.BlockSpec((bq, D), lambda i, b=b: (b * nq + i, 0)),
                pl.BlockSpec((S, D), lambda i, b=b: (b, 0)),
                pl.BlockSpec((S, D), lambda i, b=b: (b, 0)),
                pl.BlockSpec((D, D), lambda i: (0, 0)),
                pl.BlockSpec((1, D), lambda i: (0, 0)),
            ],
            out_specs=pl.BlockSpec((bq, D), lambda i: (i, 0)),
            out_shape=jax.ShapeDtypeStruct((S, D), jnp.float32),
            scratch_shapes=[pltpu.VMEM((bq, D), jnp.bfloat16)],
        )(qp, kp, vp, Wob, bo.reshape(1, D)))
    return jnp.stack(outs)


@jax.jit
def kernel(query, key, value, Wq, bq, Wk, bk, Wv, bv, Wo, bo):
    B, S, D = query.shape
    bf = jnp.bfloat16
    scale = math.log2(math.e) / math.sqrt(D // H)

    qp, kp, vp = _qkv_proj(
        query.reshape(B * S, D),
        key.reshape(B * S, D),
        value.reshape(B * S, D),
        (Wq * scale).astype(bf), bq * scale,
        Wk.astype(bf), bk,
        Wv.astype(bf), bv,
    )

    out = _attn_out(qp, kp, vp, Wo.astype(bf), bo, B=B)
    return out


# speedup vs baseline: 2.2830x; 1.1227x over previous
import functools
import math

import jax
import jax.numpy as jnp
from jax.experimental import pallas as pl

H = 16


def _qkv_kernel(x1_ref, x2_ref, x3_ref, w1_ref, w2_ref, w3_ref,
                b1_ref, b2_ref, b3_ref, wo_ref,
                o1_ref, o2_ref, o3_ref, wob_ref):
    for x_ref, w_ref, b_ref, o_ref in (
        (x1_ref, w1_ref, b1_ref, o1_ref),
        (x2_ref, w2_ref, b2_ref, o2_ref),
        (x3_ref, w3_ref, b3_ref, o3_ref),
    ):
        xb = x_ref[...].astype(jnp.bfloat16)
        acc = jnp.dot(xb, w_ref[...], preferred_element_type=jnp.float32)
        o_ref[...] = (acc + b_ref[...]).astype(jnp.bfloat16)
    wob_ref[...] = wo_ref[...].astype(jnp.bfloat16)


def _qkv_proj(q2, k2, v2, Wqs, bqs, Wk, bk, Wv, bv, Wo, bm=256):
    M, K = q2.shape
    N = Wqs.shape[1]
    g = M // bm
    ws = N // g
    bf = jnp.bfloat16
    blk_x = pl.BlockSpec((bm, K), lambda i: (i, 0))
    blk_w = pl.BlockSpec((K, N), lambda i: (0, 0))
    blk_b = pl.BlockSpec((1, N), lambda i: (0, 0))
    blk_ws = pl.BlockSpec((ws, N), lambda i: (i, 0))
    blk_o = pl.BlockSpec((bm, N), lambda i: (i, 0))
    out = jax.ShapeDtypeStruct((M, N), bf)
    return pl.pallas_call(
        _qkv_kernel,
        grid=(g,),
        in_specs=[blk_x, blk_x, blk_x, blk_w, blk_w, blk_w,
                  blk_b, blk_b, blk_b, blk_ws],
        out_specs=[blk_o, blk_o, blk_o, blk_ws],
        out_shape=[out, out, out, jax.ShapeDtypeStruct((K, N), bf)],
    )(q2, k2, v2, Wqs, Wk, Wv,
      bqs.reshape(1, N), bk.reshape(1, N), bv.reshape(1, N), Wo)


def _attn_out_kernel(q_ref, k_ref, v_ref, wo_ref, bo_ref, o_ref, ctx_ref):
    D = q_ref.shape[1]
    S = k_ref.shape[0]
    DK = D // H
    C = 256
    nc = S // C
    tokens = []
    for h in range(H):
        lo, hi = h * DK, (h + 1) * DK
        qh = q_ref[:, lo:hi]
        if h >= 2:
            qh = (qh.astype(jnp.float32)
                  + jnp.minimum(tokens[h - 2], 0.0)).astype(jnp.bfloat16)
        m = l = acc = None
        for j in range(nc):
            kj = k_ref[j * C:(j + 1) * C, lo:hi]
            s = jax.lax.dot_general(
                qh, kj, (((1,), (1,)), ((), ())),
                preferred_element_type=jnp.float32,
            )
            mj = jnp.max(s, axis=-1, keepdims=True)
            mnew = mj if m is None else jnp.maximum(m, mj)
            p = jnp.exp2(s - mnew)
            lj = jnp.sum(p, axis=-1, keepdims=True)
            vj = v_ref[j * C:(j + 1) * C, lo:hi]
            dj = jnp.dot(p.astype(jnp.bfloat16), vj,
                         preferred_element_type=jnp.float32)
            if m is None:
                l, acc = lj, dj
            else:
                alpha = jnp.exp2(m - mnew)
                l = l * alpha + lj
                acc = acc * alpha + dj
            m = mnew
        ctx_ref[:, lo:hi] = (acc / l).astype(jnp.bfloat16)
        tokens.append(l)
    o_ref[...] = (
        jnp.dot(ctx_ref[...], wo_ref[...], preferred_element_type=jnp.float32)
        + bo_ref[...]
    )


def _attn_out(qp, kp, vp, Wob, bo, B=2, bq=512):
    from jax.experimental.pallas import tpu as pltpu

    BS, D = qp.shape
    S = BS // B
    nq = S // bq
    outs = []
    for b in range(B):
        outs.append(pl.pallas_call(
            _attn_out_kernel,
            grid=(nq,),
            in_specs=[
                pl.BlockSpec((bq, D), lambda i, b=b: (b * nq + i, 0)),
                pl.BlockSpec((S, D), lambda i, b=b: (b, 0)),
                pl.BlockSpec((S, D), lambda i, b=b: (b, 0)),
                pl.BlockSpec((D, D), lambda i: (0, 0)),
                pl.BlockSpec((1, D), lambda i: (0, 0)),
            ],
            out_specs=pl.BlockSpec((bq, D), lambda i: (i, 0)),
            out_shape=jax.ShapeDtypeStruct((S, D), jnp.float32),
            scratch_shapes=[pltpu.VMEM((bq, D), jnp.bfloat16)],
        )(qp, kp, vp, Wob, bo.reshape(1, D)))
    return jnp.stack(outs)


@jax.jit
def kernel(query, key, value, Wq, bq, Wk, bk, Wv, bv, Wo, bo):
    B, S, D = query.shape
    bf = jnp.bfloat16
    scale = math.log2(math.e) / math.sqrt(D // H)

    qp, kp, vp, Wob = _qkv_proj(
        query.reshape(B * S, D),
        key.reshape(B * S, D),
        value.reshape(B * S, D),
        (Wq * scale).astype(bf), bq * scale,
        Wk.astype(bf), bk,
        Wv.astype(bf), bv,
        Wo,
    )

    out = _attn_out(qp, kp, vp, Wob, bo, B=B)
    return out
